# baseline (speedup 1.0000x reference)
import jax
import jax.numpy as jnp
from jax import lax
from jax.experimental import pallas as pl
from jax.experimental.pallas import tpu as pltpu
from jax.experimental.pallas import tpu_sc as plsc

D = 64
L = 200
LANE = 16
NJ = D // LANE
R = 8
CT = 128

_info = plsc.get_sparse_core_info()
NC = _info.num_cores
NS = _info.num_subcores
NW = NC * NS


def _tr_body(embt_hbm, out_hbm, buf0, buf1, st0, st1, *sems):
    sem_i = sems[0:2]
    sem_o = sems[2:4]
    buf = (buf0, buf1)
    st = (st0, st1)
    v = embt_hbm.shape[1]
    nfull = v // CT
    maxc = nfull - 1
    per_w = -(-nfull // NW)
    niter = (per_w + 1) // 2
    wid = lax.axis_index("s") * NC + lax.axis_index("c")

    def fire_in(c, u):
        pltpu.async_copy(
            embt_hbm.at[pl.ds(0, D), pl.ds(c * CT, CT)], buf[u], sem_i[u])

    def wait_in(u):
        pltpu.make_async_copy(
            embt_hbm.at[pl.ds(0, D), pl.ds(0, CT)], buf[u], sem_i[u]).wait()

    def fire_out(c, u):
        pltpu.async_copy(
            st[u], out_hbm.at[pl.ds(c * CT * D, CT * D)], sem_o[u])

    def wait_out(u):
        pltpu.make_async_copy(
            st[u], out_hbm.at[pl.ds(0, CT * D)], sem_o[u]).wait()

    rvec = [LANE * k + lax.iota(jnp.int32, LANE) for k in range(NJ)]

    def transpose(u):
        def col_body(col, carry):
            cvec = jnp.broadcast_to(col, (LANE,))
            for k in range(NJ):
                val = plsc.load_gather(buf[u], [rvec[k], cvec])
                st[u][pl.ds(col * D + LANE * k, LANE)] = val
            return carry

        lax.fori_loop(0, CT, col_body, 0)

    def cid(n):
        return jnp.minimum(wid + NW * n, maxc)

    fire_in(cid(0), 0)
    fire_in(cid(1), 1)

    def body(t, carry):
        for u in range(2):
            c = cid(2 * t + u)
            wait_in(u)

            @pl.when(t > 0)
            def _():
                wait_out(u)

            transpose(u)
            fire_out(c, u)
            fire_in(cid(2 * (t + 1) + u), u)
        return carry

    lax.fori_loop(0, niter, body, 0)
    for u in range(2):
        wait_in(u)
        wait_out(u)


def _transpose_table(embt):
    v = embt.shape[1]
    mesh = plsc.VectorSubcoreMesh(core_axis_name="c", subcore_axis_name="s")
    scratch = (
        [pltpu.VMEM((D, CT), jnp.float32) for _ in range(2)]
        + [pltpu.VMEM((CT * D,), jnp.float32) for _ in range(2)]
        + [pltpu.SemaphoreType.DMA for _ in range(4)]
    )
    return pl.kernel(
        _tr_body,
        out_type=jax.ShapeDtypeStruct((v * D,), jnp.float32),
        mesh=mesh,
        scratch_types=scratch,
    )(embt)


TB = 512


def _tr_tc_body(in_ref, out_ref):
    t = lax.dot_general(
        in_ref[...], jnp.eye(D, dtype=jnp.float32),
        (((0,), (0,)), ((), ())),
        preferred_element_type=jnp.float32)
    out_ref[...] = jnp.concatenate(
        [t[:TB // 2, :], t[TB // 2:, :]], axis=1)


def _transpose_table_tc(embt):
    v = embt.shape[1]
    grid = v // TB
    return pl.pallas_call(
        _tr_tc_body,
        grid=(grid,),
        in_specs=[pl.BlockSpec((D, TB), lambda i: (0, i))],
        out_specs=pl.BlockSpec((TB // 2, 2 * D), lambda i: (i, 0)),
        out_shape=jax.ShapeDtypeStruct((v // 2, 2 * D), jnp.float32),
    )(embt)


def _reduce_rows(rows_ref):
    def body(l, acc):
        a = tuple(
            acc[j] + rows_ref[2 * l, pl.ds(LANE * j, LANE)]
            for j in range(NJ))
        b = tuple(
            acc[NJ + j] + rows_ref[2 * l + 1, pl.ds(LANE * j, LANE)]
            for j in range(NJ))
        return a + b

    acc = lax.fori_loop(
        0, L // 2, body,
        tuple(jnp.zeros((LANE,), jnp.float32) for _ in range(2 * NJ)))
    return tuple(acc[j] + acc[NJ + j] for j in range(NJ))


def _pool_body(xf_hbm, emb_hbm, out_hbm, *scr):
    idx = scr[0:R]
    rows = scr[R:2 * R]
    out_v = scr[2 * R]
    sem_i = scr[2 * R + 1:3 * R + 1]
    sem_g = scr[3 * R + 1:4 * R + 1]

    bpw = out_hbm.shape[0] // NW
    niter = bpw // R
    wid = lax.axis_index("s") * NC + lax.axis_index("c")
    base = wid * bpw
    maxid = base + bpw - 1

    def stage(g, u):
        pltpu.async_copy(xf_hbm.at[pl.ds(g * L, L)], idx[u], sem_i[u])

    def wait_stage(u):
        pltpu.make_async_copy(
            xf_hbm.at[pl.ds(0, L)], idx[u], sem_i[u]).wait()

    def gather(u):
        pltpu.async_copy(emb_hbm.at[idx[u]], rows[u], sem_g[u])

    def wait_gather(u):
        pltpu.make_async_copy(emb_hbm.at[idx[u]], rows[u], sem_g[u]).wait()

    for u in range(R):
        stage(base + u, u)
    for u in range(R):
        wait_stage(u)
        gather(u)

    def body(t, carry):
        tbase = base + t * R
        for u in range(R):
            wait_gather(u)
            stage(jnp.minimum(tbase + R + u, maxid), u)
            acc = _reduce_rows(rows[u])
            for j in range(NJ):
                out_v[u, pl.ds(LANE * j, LANE)] = acc[j]
            wait_stage(u)
            gather(u)
        pltpu.sync_copy(out_v, out_hbm.at[pl.ds(tbase, R)])
        return carry

    lax.fori_loop(0, niter, body, 0)
    for u in range(R):
        wait_gather(u)


def _pooled_sums(xf, emb, b):
    mesh = plsc.VectorSubcoreMesh(core_axis_name="c", subcore_axis_name="s")
    scratch = (
        [pltpu.VMEM((L,), jnp.int32) for _ in range(R)]
        + [pltpu.VMEM((L, D), jnp.float32) for _ in range(R)]
        + [pltpu.VMEM((R, D), jnp.float32)]
        + [pltpu.SemaphoreType.DMA for _ in range(2 * R)]
    )
    return pl.kernel(
        _pool_body,
        out_type=jax.ShapeDtypeStruct((b, D), jnp.float32),
        mesh=mesh,
        compiler_params=pltpu.CompilerParams(use_tc_tiling_on_sc=False),
        scratch_types=scratch,
    )(xf, emb)


def _mlp_body(s_ref, x_ref, e0_ref, w1_ref, b1_ref, w2_ref, b2_ref,
              w3t_ref, b3_ref, out_ref):
    count0 = jnp.sum((x_ref[...] == 0).astype(jnp.float32), axis=1,
                     keepdims=True)
    pooled = (s_ref[...] - count0 * e0_ref[...]) * jnp.float32(1.0 / L)
    h1 = jnp.maximum(
        jnp.dot(pooled, w1_ref[...], preferred_element_type=jnp.float32)
        + b1_ref[...], 0.0)
    h2 = jnp.maximum(
        jnp.dot(h1, w2_ref[...], preferred_element_type=jnp.float32)
        + b2_ref[...], 0.0)
    o = jnp.sum(h2 * w3t_ref[...], axis=1, keepdims=True) + b3_ref[...]
    out_ref[...] = jax.nn.sigmoid(o)


def _mlp(sums, x, emb0, W1, b1, W2, b2, W3, b3):
    b = sums.shape[0]
    blk = 1024
    grid = b // blk
    h = W2.shape[1]
    out = pl.pallas_call(
        _mlp_body,
        grid=(grid,),
        in_specs=[
            pl.BlockSpec((blk, D), lambda i: (i, 0)),
            pl.BlockSpec((blk, L), lambda i: (i, 0)),
            pl.BlockSpec((1, D), lambda i: (0, 0)),
            pl.BlockSpec((D, D), lambda i: (0, 0)),
            pl.BlockSpec((1, D), lambda i: (0, 0)),
            pl.BlockSpec((D, h), lambda i: (0, 0)),
            pl.BlockSpec((1, h), lambda i: (0, 0)),
            pl.BlockSpec((1, h), lambda i: (0, 0)),
            pl.BlockSpec((1, 1), lambda i: (0, 0)),
        ],
        out_specs=pl.BlockSpec((blk, 1), lambda i: (i, 0)),
        out_shape=jax.ShapeDtypeStruct((b, 1), jnp.float32),
    )(sums, x, emb0, W1, b1.reshape(1, -1), W2, b2.reshape(1, -1),
      W3.reshape(1, -1), b3.reshape(1, 1))
    return out[:, 0]


def kernel(x, emb, W1, b1, W2, b2, W3, b3):
    v = emb.shape[0]
    paired = _transpose_table_tc(emb.T)
    tail = (v // TB) * TB
    if tail < v:
        paired = paired.at[tail // 2:].set(emb[tail:].reshape(-1, 2 * D))
    emb2 = paired.reshape(-1).reshape(emb.shape)
    q = x % TB
    h = q // (TB // 2)
    xt = jnp.where(x < tail, (x - q) + 2 * (q - h * (TB // 2)) + h, x)
    sums = _pooled_sums(xt.reshape(-1), emb2, x.shape[0])
    return _mlp(sums, x, emb[0:1, :], W1, b1, W2, b2, W3, b3)


# speedup vs baseline: 2.4762x; 2.4762x over previous
import jax
import jax.numpy as jnp
from jax import lax
from jax.experimental import pallas as pl
from jax.experimental.pallas import tpu as pltpu
from jax.experimental.pallas import tpu_sc as plsc

D = 64
L = 200
LANE = 16
NJ = D // LANE
R = 8
CT = 128

_info = plsc.get_sparse_core_info()
NC = _info.num_cores
NS = _info.num_subcores
NW = NC * NS


def _tr_body(embt_hbm, out_hbm, buf0, buf1, st0, st1, *sems):
    sem_i = sems[0:2]
    sem_o = sems[2:4]
    buf = (buf0, buf1)
    st = (st0, st1)
    v = embt_hbm.shape[1]
    nfull = v // CT
    maxc = nfull - 1
    per_w = -(-nfull // NW)
    niter = (per_w + 1) // 2
    wid = lax.axis_index("s") * NC + lax.axis_index("c")

    def fire_in(c, u):
        pltpu.async_copy(
            embt_hbm.at[pl.ds(0, D), pl.ds(c * CT, CT)], buf[u], sem_i[u])

    def wait_in(u):
        pltpu.make_async_copy(
            embt_hbm.at[pl.ds(0, D), pl.ds(0, CT)], buf[u], sem_i[u]).wait()

    def fire_out(c, u):
        pltpu.async_copy(
            st[u], out_hbm.at[pl.ds(c * CT * D, CT * D)], sem_o[u])

    def wait_out(u):
        pltpu.make_async_copy(
            st[u], out_hbm.at[pl.ds(0, CT * D)], sem_o[u]).wait()

    rvec = [LANE * k + lax.iota(jnp.int32, LANE) for k in range(NJ)]

    def transpose(u):
        def col_body(col, carry):
            cvec = jnp.broadcast_to(col, (LANE,))
            for k in range(NJ):
                val = plsc.load_gather(buf[u], [rvec[k], cvec])
                st[u][pl.ds(col * D + LANE * k, LANE)] = val
            return carry

        lax.fori_loop(0, CT, col_body, 0)

    def cid(n):
        return jnp.minimum(wid + NW * n, maxc)

    fire_in(cid(0), 0)
    fire_in(cid(1), 1)

    def body(t, carry):
        for u in range(2):
            c = cid(2 * t + u)
            wait_in(u)

            @pl.when(t > 0)
            def _():
                wait_out(u)

            transpose(u)
            fire_out(c, u)
            fire_in(cid(2 * (t + 1) + u), u)
        return carry

    lax.fori_loop(0, niter, body, 0)
    for u in range(2):
        wait_in(u)
        wait_out(u)


def _transpose_table(embt):
    v = embt.shape[1]
    mesh = plsc.VectorSubcoreMesh(core_axis_name="c", subcore_axis_name="s")
    scratch = (
        [pltpu.VMEM((D, CT), jnp.float32) for _ in range(2)]
        + [pltpu.VMEM((CT * D,), jnp.float32) for _ in range(2)]
        + [pltpu.SemaphoreType.DMA for _ in range(4)]
    )
    return pl.kernel(
        _tr_body,
        out_type=jax.ShapeDtypeStruct((v * D,), jnp.float32),
        mesh=mesh,
        scratch_types=scratch,
    )(embt)


TB = 8192


def _tr_tc_body(in_ref, out_ref):
    t = lax.dot_general(
        in_ref[...], jnp.eye(D, dtype=jnp.float32),
        (((0,), (0,)), ((), ())),
        preferred_element_type=jnp.float32)
    out_ref[...] = jnp.concatenate(
        [t[:TB // 2, :], t[TB // 2:, :]], axis=1)


def _transpose_table_tc(embt):
    v = embt.shape[1]
    grid = v // TB
    return pl.pallas_call(
        _tr_tc_body,
        grid=(grid,),
        in_specs=[pl.BlockSpec((D, TB), lambda i: (0, i))],
        out_specs=pl.BlockSpec((TB // 2, 2 * D), lambda i: (i, 0)),
        out_shape=jax.ShapeDtypeStruct((v // 2, 2 * D), jnp.float32),
    )(embt)


def _reduce_rows(rows_ref):
    def body(l, acc):
        a = tuple(
            acc[j] + rows_ref[2 * l, pl.ds(LANE * j, LANE)]
            for j in range(NJ))
        b = tuple(
            acc[NJ + j] + rows_ref[2 * l + 1, pl.ds(LANE * j, LANE)]
            for j in range(NJ))
        return a + b

    acc = lax.fori_loop(
        0, L // 2, body,
        tuple(jnp.zeros((LANE,), jnp.float32) for _ in range(2 * NJ)))
    return tuple(acc[j] + acc[NJ + j] for j in range(NJ))


def _pool_body(xf_hbm, emb_hbm, out_hbm, *scr):
    idx = scr[0:R]
    rows = scr[R:2 * R]
    out_v = scr[2 * R]
    sem_i = scr[2 * R + 1:3 * R + 1]
    sem_g = scr[3 * R + 1:4 * R + 1]

    bpw = out_hbm.shape[0] // NW
    niter = bpw // R
    wid = lax.axis_index("s") * NC + lax.axis_index("c")
    base = wid * bpw
    maxid = base + bpw - 1

    def stage(g, u):
        pltpu.async_copy(xf_hbm.at[pl.ds(g * L, L)], idx[u], sem_i[u])

    def wait_stage(u):
        pltpu.make_async_copy(
            xf_hbm.at[pl.ds(0, L)], idx[u], sem_i[u]).wait()

    def gather(u):
        pltpu.async_copy(emb_hbm.at[idx[u]], rows[u], sem_g[u])

    def wait_gather(u):
        pltpu.make_async_copy(emb_hbm.at[idx[u]], rows[u], sem_g[u]).wait()

    for u in range(R):
        stage(base + u, u)
    for u in range(R):
        wait_stage(u)
        gather(u)

    def body(t, carry):
        tbase = base + t * R
        for u in range(R):
            wait_gather(u)
            stage(jnp.minimum(tbase + R + u, maxid), u)
            acc = _reduce_rows(rows[u])
            for j in range(NJ):
                out_v[u, pl.ds(LANE * j, LANE)] = acc[j]
            wait_stage(u)
            gather(u)
        pltpu.sync_copy(out_v, out_hbm.at[pl.ds(tbase, R)])
        return carry

    lax.fori_loop(0, niter, body, 0)
    for u in range(R):
        wait_gather(u)


def _pooled_sums(xf, emb, b):
    mesh = plsc.VectorSubcoreMesh(core_axis_name="c", subcore_axis_name="s")
    scratch = (
        [pltpu.VMEM((L,), jnp.int32) for _ in range(R)]
        + [pltpu.VMEM((L, D), jnp.float32) for _ in range(R)]
        + [pltpu.VMEM((R, D), jnp.float32)]
        + [pltpu.SemaphoreType.DMA for _ in range(2 * R)]
    )
    return pl.kernel(
        _pool_body,
        out_type=jax.ShapeDtypeStruct((b, D), jnp.float32),
        mesh=mesh,
        compiler_params=pltpu.CompilerParams(use_tc_tiling_on_sc=False),
        scratch_types=scratch,
    )(xf, emb)


def _mlp_body(s_ref, x_ref, e0_ref, w1_ref, b1_ref, w2_ref, b2_ref,
              w3t_ref, b3_ref, out_ref):
    count0 = jnp.sum((x_ref[...] == 0).astype(jnp.float32), axis=1,
                     keepdims=True)
    pooled = (s_ref[...] - count0 * e0_ref[...]) * jnp.float32(1.0 / L)
    h1 = jnp.maximum(
        jnp.dot(pooled, w1_ref[...], preferred_element_type=jnp.float32)
        + b1_ref[...], 0.0)
    h2 = jnp.maximum(
        jnp.dot(h1, w2_ref[...], preferred_element_type=jnp.float32)
        + b2_ref[...], 0.0)
    o = jnp.sum(h2 * w3t_ref[...], axis=1, keepdims=True) + b3_ref[...]
    out_ref[...] = jax.nn.sigmoid(o)


def _mlp(sums, x, emb0, W1, b1, W2, b2, W3, b3):
    b = sums.shape[0]
    blk = 1024
    grid = b // blk
    h = W2.shape[1]
    out = pl.pallas_call(
        _mlp_body,
        grid=(grid,),
        in_specs=[
            pl.BlockSpec((blk, D), lambda i: (i, 0)),
            pl.BlockSpec((blk, L), lambda i: (i, 0)),
            pl.BlockSpec((1, D), lambda i: (0, 0)),
            pl.BlockSpec((D, D), lambda i: (0, 0)),
            pl.BlockSpec((1, D), lambda i: (0, 0)),
            pl.BlockSpec((D, h), lambda i: (0, 0)),
            pl.BlockSpec((1, h), lambda i: (0, 0)),
            pl.BlockSpec((1, h), lambda i: (0, 0)),
            pl.BlockSpec((1, 1), lambda i: (0, 0)),
        ],
        out_specs=pl.BlockSpec((blk, 1), lambda i: (i, 0)),
        out_shape=jax.ShapeDtypeStruct((b, 1), jnp.float32),
    )(sums, x, emb0, W1, b1.reshape(1, -1), W2, b2.reshape(1, -1),
      W3.reshape(1, -1), b3.reshape(1, 1))
    return out[:, 0]


def kernel(x, emb, W1, b1, W2, b2, W3, b3):
    v = emb.shape[0]
    paired = _transpose_table_tc(emb.T)
    tail = (v // TB) * TB
    if tail < v:
        paired = paired.at[tail // 2:].set(emb[tail:].reshape(-1, 2 * D))
    emb2 = paired.reshape(-1).reshape(emb.shape)
    q = x % TB
    h = q // (TB // 2)
    xt = jnp.where(x < tail, (x - q) + 2 * (q - h * (TB // 2)) + h, x)
    sums = _pooled_sums(xt.reshape(-1), emb2, x.shape[0])
    return _mlp(sums, x, emb[0:1, :], W1, b1, W2, b2, W3, b3)


# speedup vs baseline: 2.6072x; 1.0529x over previous
import jax
import jax.numpy as jnp
from jax import lax
from jax.experimental import pallas as pl
from jax.experimental.pallas import tpu as pltpu
from jax.experimental.pallas import tpu_sc as plsc

D = 64
L = 200
LANE = 16
NJ = D // LANE
R = 8
CT = 128

_info = plsc.get_sparse_core_info()
NC = _info.num_cores
NS = _info.num_subcores
NW = NC * NS


def _tr_body(embt_hbm, out_hbm, buf0, buf1, st0, st1, *sems):
    sem_i = sems[0:2]
    sem_o = sems[2:4]
    buf = (buf0, buf1)
    st = (st0, st1)
    v = embt_hbm.shape[1]
    nfull = v // CT
    maxc = nfull - 1
    per_w = -(-nfull // NW)
    niter = (per_w + 1) // 2
    wid = lax.axis_index("s") * NC + lax.axis_index("c")

    def fire_in(c, u):
        pltpu.async_copy(
            embt_hbm.at[pl.ds(0, D), pl.ds(c * CT, CT)], buf[u], sem_i[u])

    def wait_in(u):
        pltpu.make_async_copy(
            embt_hbm.at[pl.ds(0, D), pl.ds(0, CT)], buf[u], sem_i[u]).wait()

    def fire_out(c, u):
        pltpu.async_copy(
            st[u], out_hbm.at[pl.ds(c * CT * D, CT * D)], sem_o[u])

    def wait_out(u):
        pltpu.make_async_copy(
            st[u], out_hbm.at[pl.ds(0, CT * D)], sem_o[u]).wait()

    rvec = [LANE * k + lax.iota(jnp.int32, LANE) for k in range(NJ)]

    def transpose(u):
        def col_body(col, carry):
            cvec = jnp.broadcast_to(col, (LANE,))
            for k in range(NJ):
                val = plsc.load_gather(buf[u], [rvec[k], cvec])
                st[u][pl.ds(col * D + LANE * k, LANE)] = val
            return carry

        lax.fori_loop(0, CT, col_body, 0)

    def cid(n):
        return jnp.minimum(wid + NW * n, maxc)

    fire_in(cid(0), 0)
    fire_in(cid(1), 1)

    def body(t, carry):
        for u in range(2):
            c = cid(2 * t + u)
            wait_in(u)

            @pl.when(t > 0)
            def _():
                wait_out(u)

            transpose(u)
            fire_out(c, u)
            fire_in(cid(2 * (t + 1) + u), u)
        return carry

    lax.fori_loop(0, niter, body, 0)
    for u in range(2):
        wait_in(u)
        wait_out(u)


def _transpose_table(embt):
    v = embt.shape[1]
    mesh = plsc.VectorSubcoreMesh(core_axis_name="c", subcore_axis_name="s")
    scratch = (
        [pltpu.VMEM((D, CT), jnp.float32) for _ in range(2)]
        + [pltpu.VMEM((CT * D,), jnp.float32) for _ in range(2)]
        + [pltpu.SemaphoreType.DMA for _ in range(4)]
    )
    return pl.kernel(
        _tr_body,
        out_type=jax.ShapeDtypeStruct((v * D,), jnp.float32),
        mesh=mesh,
        scratch_types=scratch,
    )(embt)


TB = 16384


def _tr_tc_body(in_ref, out_ref):
    t = lax.dot_general(
        in_ref[...], jnp.eye(D, dtype=jnp.float32),
        (((0,), (0,)), ((), ())),
        preferred_element_type=jnp.float32)
    out_ref[...] = jnp.concatenate(
        [t[:TB // 2, :], t[TB // 2:, :]], axis=1)


def _transpose_table_tc(embt):
    v = embt.shape[1]
    grid = v // TB
    return pl.pallas_call(
        _tr_tc_body,
        grid=(grid,),
        in_specs=[pl.BlockSpec((D, TB), lambda i: (0, i))],
        out_specs=pl.BlockSpec((TB // 2, 2 * D), lambda i: (i, 0)),
        out_shape=jax.ShapeDtypeStruct((v // 2, 2 * D), jnp.float32),
    )(embt)


def _reduce_rows(rows_ref):
    def body(l, acc):
        a = tuple(
            acc[j] + rows_ref[2 * l, pl.ds(LANE * j, LANE)]
            for j in range(NJ))
        b = tuple(
            acc[NJ + j] + rows_ref[2 * l + 1, pl.ds(LANE * j, LANE)]
            for j in range(NJ))
        return a + b

    acc = lax.fori_loop(
        0, L // 2, body,
        tuple(jnp.zeros((LANE,), jnp.float32) for _ in range(2 * NJ)))
    return tuple(acc[j] + acc[NJ + j] for j in range(NJ))


def _pool_body(xf_hbm, emb_hbm, out_hbm, *scr):
    idx = scr[0:R]
    rows = scr[R:2 * R]
    out_v = scr[2 * R]
    sem_i = scr[2 * R + 1:3 * R + 1]
    sem_g = scr[3 * R + 1:4 * R + 1]

    bpw = out_hbm.shape[0] // NW
    niter = bpw // R
    wid = lax.axis_index("s") * NC + lax.axis_index("c")
    base = wid * bpw
    maxid = base + bpw - 1

    def stage(g, u):
        pltpu.async_copy(xf_hbm.at[pl.ds(g * L, L)], idx[u], sem_i[u])

    def wait_stage(u):
        pltpu.make_async_copy(
            xf_hbm.at[pl.ds(0, L)], idx[u], sem_i[u]).wait()

    def gather(u):
        pltpu.async_copy(emb_hbm.at[idx[u]], rows[u], sem_g[u])

    def wait_gather(u):
        pltpu.make_async_copy(emb_hbm.at[idx[u]], rows[u], sem_g[u]).wait()

    for u in range(R):
        stage(base + u, u)
    for u in range(R):
        wait_stage(u)
        gather(u)

    def body(t, carry):
        tbase = base + t * R
        for u in range(R):
            wait_gather(u)
            stage(jnp.minimum(tbase + R + u, maxid), u)
            acc = _reduce_rows(rows[u])
            for j in range(NJ):
                out_v[u, pl.ds(LANE * j, LANE)] = acc[j]
            wait_stage(u)
            gather(u)
        pltpu.sync_copy(out_v, out_hbm.at[pl.ds(tbase, R)])
        return carry

    lax.fori_loop(0, niter, body, 0)
    for u in range(R):
        wait_gather(u)


def _pooled_sums(xf, emb, b):
    mesh = plsc.VectorSubcoreMesh(core_axis_name="c", subcore_axis_name="s")
    scratch = (
        [pltpu.VMEM((L,), jnp.int32) for _ in range(R)]
        + [pltpu.VMEM((L, D), jnp.float32) for _ in range(R)]
        + [pltpu.VMEM((R, D), jnp.float32)]
        + [pltpu.SemaphoreType.DMA for _ in range(2 * R)]
    )
    return pl.kernel(
        _pool_body,
        out_type=jax.ShapeDtypeStruct((b, D), jnp.float32),
        mesh=mesh,
        compiler_params=pltpu.CompilerParams(use_tc_tiling_on_sc=False),
        scratch_types=scratch,
    )(xf, emb)


def _mlp_body(s_ref, x_ref, e0_ref, w1_ref, b1_ref, w2_ref, b2_ref,
              w3t_ref, b3_ref, out_ref):
    count0 = jnp.sum((x_ref[...] == 0).astype(jnp.float32), axis=1,
                     keepdims=True)
    pooled = (s_ref[...] - count0 * e0_ref[...]) * jnp.float32(1.0 / L)
    h1 = jnp.maximum(
        jnp.dot(pooled, w1_ref[...], preferred_element_type=jnp.float32)
        + b1_ref[...], 0.0)
    h2 = jnp.maximum(
        jnp.dot(h1, w2_ref[...], preferred_element_type=jnp.float32)
        + b2_ref[...], 0.0)
    o = jnp.sum(h2 * w3t_ref[...], axis=1, keepdims=True) + b3_ref[...]
    out_ref[...] = jax.nn.sigmoid(o)


def _mlp(sums, x, emb0, W1, b1, W2, b2, W3, b3):
    b = sums.shape[0]
    blk = 1024
    grid = b // blk
    h = W2.shape[1]
    out = pl.pallas_call(
        _mlp_body,
        grid=(grid,),
        in_specs=[
            pl.BlockSpec((blk, D), lambda i: (i, 0)),
            pl.BlockSpec((blk, L), lambda i: (i, 0)),
            pl.BlockSpec((1, D), lambda i: (0, 0)),
            pl.BlockSpec((D, D), lambda i: (0, 0)),
            pl.BlockSpec((1, D), lambda i: (0, 0)),
            pl.BlockSpec((D, h), lambda i: (0, 0)),
            pl.BlockSpec((1, h), lambda i: (0, 0)),
            pl.BlockSpec((1, h), lambda i: (0, 0)),
            pl.BlockSpec((1, 1), lambda i: (0, 0)),
        ],
        out_specs=pl.BlockSpec((blk, 1), lambda i: (i, 0)),
        out_shape=jax.ShapeDtypeStruct((b, 1), jnp.float32),
    )(sums, x, emb0, W1, b1.reshape(1, -1), W2, b2.reshape(1, -1),
      W3.reshape(1, -1), b3.reshape(1, 1))
    return out[:, 0]


def kernel(x, emb, W1, b1, W2, b2, W3, b3):
    v = emb.shape[0]
    paired = _transpose_table_tc(emb.T)
    tail = (v // TB) * TB
    if tail < v:
        paired = paired.at[tail // 2:].set(emb[tail:].reshape(-1, 2 * D))
    emb2 = paired.reshape(-1).reshape(emb.shape)
    q = x % TB
    h = q // (TB // 2)
    xt = jnp.where(x < tail, (x - q) + 2 * (q - h * (TB // 2)) + h, x)
    sums = _pooled_sums(xt.reshape(-1), emb2, x.shape[0])
    return _mlp(sums, x, emb[0:1, :], W1, b1, W2, b2, W3, b3)
